# Initial kernel scaffold; baseline (speedup 1.0000x reference)
import functools

import jax
import jax.numpy as jnp
from jax import lax
from jax.experimental import pallas as pl
from jax.experimental.pallas import tpu as pltpu
from jax.experimental.pallas import tpu_sc as plsc

NC = 2
NS = 16
NW = NC * NS

F32 = jnp.float32


def _make_degree_kernel(NP, C, B):
  mesh = plsc.VectorSubcoreMesh(core_axis_name="c", subcore_axis_name="s")
  rows_per_tile = NP // NS
  ZR = 64
  n_zchunks = rows_per_tile // ZR

  @functools.partial(
      pl.kernel,
      mesh=mesh,
      out_type=(
          jax.ShapeDtypeStruct((NC, NP, 16), F32),
          jax.ShapeDtypeStruct((NC, NP, 16), F32),
      ),
      scratch_types=[
          pltpu.VMEM_SHARED((NP, 16), F32),
          pltpu.VMEM_SHARED((NP, 16), F32),
          pltpu.VMEM((C, B), jnp.int32),
          pltpu.VMEM((C, B), jnp.int32),
          pltpu.VMEM((B, 16), F32),
          pltpu.VMEM((64, 16), F32),
          pltpu.SemaphoreType.DMA,
      ],
  )
  def deg_kernel(src3, dst3, out_s, out_d, acc_s, acc_d, sidx, didx, ones_v,
                 zb, sem):
    cid = lax.axis_index("c")
    tid = lax.axis_index("s")
    wid = cid * NS + tid
    row0 = tid * rows_per_tile

    pltpu.async_copy(src3.at[wid], sidx, sem).wait()
    pltpu.async_copy(dst3.at[wid], didx, sem).wait()

    def fill_ones(i, _):
      ones_v[i] = jnp.ones((16,), F32)
      return 0
    lax.fori_loop(0, B, fill_ones, 0)

    def fill_zeros(i, _):
      zb[i] = jnp.zeros((16,), F32)
      return 0
    lax.fori_loop(0, 64, fill_zeros, 0)

    for k in range(n_zchunks):
      pltpu.sync_copy(zb, acc_s.at[pl.ds(row0 + k * 64, 64)])
      pltpu.sync_copy(zb, acc_d.at[pl.ds(row0 + k * 64, 64)])
    plsc.subcore_barrier()

    def body(j, _):
      pltpu.sync_copy(ones_v, acc_s.at[sidx.at[j]], add=True)
      pltpu.sync_copy(ones_v, acc_d.at[didx.at[j]], add=True)
      return 0
    lax.fori_loop(0, C, body, 0)
    plsc.subcore_barrier()

    pltpu.sync_copy(acc_s.at[pl.ds(row0, rows_per_tile)],
                    out_s.at[cid, pl.ds(row0, rows_per_tile)])
    pltpu.sync_copy(acc_d.at[pl.ds(row0, rows_per_tile)],
                    out_d.at[cid, pl.ds(row0, rows_per_tile)])

  return deg_kernel


def _make_scatter_kernel(N, D, NP, C, B):
  mesh = plsc.VectorSubcoreMesh(core_axis_name="c", subcore_axis_name="s")
  rows_per_tile = NP // NS
  ZR = 64
  n_zchunks = rows_per_tile // ZR
  d_vecs = D // 16

  @functools.partial(
      pl.kernel,
      mesh=mesh,
      out_type=jax.ShapeDtypeStruct((NC, NP, D), F32),
      scratch_types=[
          pltpu.VMEM_SHARED((NP, D), F32),
          pltpu.VMEM((C, B), jnp.int32),
          pltpu.VMEM((C, B), jnp.int32),
          pltpu.VMEM((B, D), F32),
          pltpu.VMEM((ZR, D), F32),
          pltpu.SemaphoreType.DMA,
      ],
  )
  def mp_kernel(z_hbm, src3, dst3, out, acc, sidx, didx, rows_v, zb, sem):
    cid = lax.axis_index("c")
    tid = lax.axis_index("s")
    wid = cid * NS + tid
    row0 = tid * rows_per_tile

    pltpu.async_copy(src3.at[wid], sidx, sem).wait()
    pltpu.async_copy(dst3.at[wid], didx, sem).wait()

    def fill_zeros(i, _):
      for c in range(d_vecs):
        zb[i, pl.ds(c * 16, 16)] = jnp.zeros((16,), F32)
      return 0
    lax.fori_loop(0, ZR, fill_zeros, 0)

    for k in range(n_zchunks):
      pltpu.sync_copy(zb, acc.at[pl.ds(row0 + k * ZR, ZR)])
    plsc.subcore_barrier()

    def body(j, _):
      pltpu.async_copy(z_hbm.at[sidx.at[j]], rows_v, sem).wait()
      pltpu.sync_copy(rows_v, acc.at[didx.at[j]], add=True)
      return 0
    lax.fori_loop(0, C, body, 0)
    plsc.subcore_barrier()

    for k in range(n_zchunks):
      pltpu.sync_copy(acc.at[pl.ds(row0 + k * ZR, ZR)],
                      out.at[cid, pl.ds(row0 + k * ZR, ZR)])

  return mp_kernel


def _norm_from_hist(hist_blk):
  deg = hist_blk[0, :, 0] + hist_blk[1, :, 0]
  return lax.rsqrt(jnp.maximum(deg, 1.0))


def _scale_matmul_body(degs_ref, x_ref, w_ref, o_ref):
  norm = _norm_from_hist(degs_ref[...])
  o_ref[...] = (x_ref[...] * norm[:, None]) @ w_ref[...]


def _layer_mid_body(p_ref, degi_ref, dego_ref, b1_ref, w2_ref, o_ref):
  norm_i = _norm_from_hist(degi_ref[...])
  norm_o = _norm_from_hist(dego_ref[...])
  agg = p_ref[0] + p_ref[1]
  h = jnp.maximum(agg * norm_i[:, None] + b1_ref[...], 0.0)
  o_ref[...] = (h * norm_o[:, None]) @ w2_ref[...]


def _layer_out_body(p_ref, degi_ref, b2_ref, o_ref):
  norm_i = _norm_from_hist(degi_ref[...])
  agg = p_ref[0] + p_ref[1]
  o_ref[...] = agg * norm_i[:, None] + b2_ref[...]


def _tc_call(body, grid, in_specs, out_shape, out_spec):
  return pl.pallas_call(
      body,
      grid=grid,
      in_specs=in_specs,
      out_specs=out_spec,
      out_shape=out_shape,
  )


def kernel(in_feat, edge_index, W1, b1, W2, b2):
  N, D_IN = in_feat.shape
  E = edge_index.shape[1]
  D_H = W1.shape[1]
  D_OUT = W2.shape[1]

  NP = ((N + 1023) // 1024) * 1024
  B = 128
  C = E // (NW * B)
  assert C * NW * B == E

  src3 = edge_index[0].reshape(NW, C, B)
  dst3 = edge_index[1].reshape(NW, C, B)

  deg_kernel = _make_degree_kernel(NP, C, B)
  hist_s, hist_d = deg_kernel(src3, dst3)

  RB = 1000
  n_blocks = N // RB
  hist_spec = pl.BlockSpec((NC, RB, 16), lambda i: (0, i, 0))
  mat_spec = pl.BlockSpec((RB, D_IN), lambda i: (i, 0))
  w_spec = pl.BlockSpec((D_IN, D_H), lambda i: (0, 0))
  b_spec = pl.BlockSpec((1, D_H), lambda i: (0, 0))
  part_spec = pl.BlockSpec((NC, RB, D_H), lambda i: (0, i, 0))

  z1 = _tc_call(
      _scale_matmul_body, (n_blocks,),
      [hist_spec, mat_spec, w_spec],
      jax.ShapeDtypeStruct((N, D_H), F32),
      pl.BlockSpec((RB, D_H), lambda i: (i, 0)),
  )(hist_s, in_feat, W1)

  mp_kernel = _make_scatter_kernel(N, D_H, NP, C, B)
  p1 = mp_kernel(z1, src3, dst3)

  z2 = _tc_call(
      _layer_mid_body, (n_blocks,),
      [part_spec, hist_spec, hist_spec, b_spec,
       pl.BlockSpec((D_H, D_OUT), lambda i: (0, 0))],
      jax.ShapeDtypeStruct((N, D_OUT), F32),
      pl.BlockSpec((RB, D_OUT), lambda i: (i, 0)),
  )(p1, hist_d, hist_s, b1.reshape(1, D_H), W2)

  p2 = mp_kernel(z2, src3, dst3)

  out = _tc_call(
      _layer_out_body, (n_blocks,),
      [pl.BlockSpec((NC, RB, D_OUT), lambda i: (0, i, 0)), hist_spec, b_spec],
      jax.ShapeDtypeStruct((N, D_OUT), F32),
      pl.BlockSpec((RB, D_OUT), lambda i: (i, 0)),
  )(p2, hist_d, b2.reshape(1, D_OUT))

  return out


# speedup vs baseline: 3.0866x; 3.0866x over previous
import functools

import jax
import jax.numpy as jnp
from jax import lax
from jax.experimental import pallas as pl
from jax.experimental.pallas import tpu as pltpu
from jax.experimental.pallas import tpu_sc as plsc

NC = 2
NS = 16
NW = NC * NS

F32 = jnp.float32
I32 = jnp.int32

ZB = 32


def _fill_myrows(myrows, row0, n_chunks):
  ivec = lax.iota(I32, 16)
  one_i = jnp.full((16,), 1, I32)

  def fr(i, _):
    z = i // 2
    off = (i % 2) * 16
    myrows[z, pl.ds(off, 16)] = ivec + one_i * (row0 + i * 16)
    return 0
  lax.fori_loop(0, n_chunks * 2, fr, 0)


def _fill_zeros(zb, d_vecs):
  def fz(i, _):
    for c in range(d_vecs):
      zb[i, pl.ds(c * 16, 16)] = jnp.zeros((16,), F32)
    return 0
  lax.fori_loop(0, ZB, fz, 0)


def _out_rows(out, cid, row0, z):
  off = pl.multiple_of(row0 + z * ZB, ZB)
  return out.at[cid, pl.ds(off, ZB)]


def _make_degree_kernel(NP, C, B, GS):
  mesh = plsc.VectorSubcoreMesh(core_axis_name="c", subcore_axis_name="s",
                                num_cores=NC, num_subcores=NS)
  rpt = NP // NS
  n_zchunks = rpt // ZB
  n_groups = C // GS

  @functools.partial(
      pl.kernel,
      mesh=mesh,
      out_type=jax.ShapeDtypeStruct((NC, NP, 128), F32),
      scratch_types=[
          pltpu.VMEM_SHARED((NP, 128), F32),
          pltpu.VMEM((GS, B), I32),
          pltpu.VMEM((GS, B), I32),
          pltpu.VMEM((n_zchunks, ZB), I32),
          pltpu.VMEM((B, 128), F32),
          pltpu.VMEM((B, 128), F32),
          pltpu.VMEM((ZB, 128), F32),
          pltpu.SemaphoreType.DMA,
      ],
  )
  def deg_kernel(src3, dst3, out, acc, sidx, didx, myrows, ones_l, ones_r,
                 zb, sem):
    cid = lax.axis_index("c")
    tid = lax.axis_index("s")
    wid = cid * NS + tid
    row0 = tid * rpt

    def fill_ones(i, _):
      for c in range(8):
        v = 1.0 if c < 4 else 0.0
        ones_l[i, pl.ds(c * 16, 16)] = jnp.full((16,), v, F32)
        ones_r[i, pl.ds(c * 16, 16)] = jnp.full((16,), 1.0 - v, F32)
      return 0
    lax.fori_loop(0, B, fill_ones, 0)

    _fill_zeros(zb, 8)
    _fill_myrows(myrows, row0, n_zchunks)

    def zero(z, _):
      pltpu.sync_copy(zb, acc.at[myrows.at[z]])
      return 0
    lax.fori_loop(0, n_zchunks, zero, 0)
    plsc.subcore_barrier()

    def group(g, _):
      goff = pl.multiple_of(g * GS, GS)
      pltpu.async_copy(src3.at[wid, pl.ds(goff, GS)], sidx, sem).wait()
      pltpu.async_copy(dst3.at[wid, pl.ds(goff, GS)], didx, sem).wait()

      def body(j, _):
        pltpu.sync_copy(ones_l, acc.at[sidx.at[j]], add=True)
        pltpu.sync_copy(ones_r, acc.at[didx.at[j]], add=True)
        return 0
      lax.fori_loop(0, GS, body, 0)
      return 0
    lax.fori_loop(0, n_groups, group, 0)
    plsc.subcore_barrier()

    def readout(z, _):
      pltpu.async_copy(acc.at[myrows.at[z]], zb, sem).wait()
      pltpu.sync_copy(zb, _out_rows(out, cid, row0, z))
      return 0
    lax.fori_loop(0, n_zchunks, readout, 0)

  return deg_kernel


def _make_scatter_kernel(D, NP, C, B, GS):
  mesh = plsc.VectorSubcoreMesh(core_axis_name="c", subcore_axis_name="s",
                                num_cores=NC, num_subcores=NS)
  rpt = NP // NS
  n_zchunks = rpt // ZB
  n_groups = C // GS

  @functools.partial(
      pl.kernel,
      mesh=mesh,
      out_type=jax.ShapeDtypeStruct((NC, NP, D), F32),
      scratch_types=[
          pltpu.VMEM_SHARED((NP, D), F32),
          pltpu.VMEM((GS, B), I32),
          pltpu.VMEM((GS, B), I32),
          pltpu.VMEM((n_zchunks, ZB), I32),
          pltpu.VMEM((B, D), F32),
          pltpu.VMEM((ZB, D), F32),
          pltpu.SemaphoreType.DMA,
      ],
  )
  def mp_kernel(z_hbm, src3, dst3, out, acc, sidx, didx, myrows, rows_v, zb,
                sem):
    cid = lax.axis_index("c")
    tid = lax.axis_index("s")
    wid = cid * NS + tid
    row0 = tid * rpt

    _fill_zeros(zb, D // 16)
    _fill_myrows(myrows, row0, n_zchunks)

    def zero(z, _):
      pltpu.sync_copy(zb, acc.at[myrows.at[z]])
      return 0
    lax.fori_loop(0, n_zchunks, zero, 0)
    plsc.subcore_barrier()

    def group(g, _):
      goff = pl.multiple_of(g * GS, GS)
      pltpu.async_copy(src3.at[wid, pl.ds(goff, GS)], sidx, sem).wait()
      pltpu.async_copy(dst3.at[wid, pl.ds(goff, GS)], didx, sem).wait()

      def body(j, _):
        pltpu.async_copy(z_hbm.at[sidx.at[j]], rows_v, sem).wait()
        pltpu.sync_copy(rows_v, acc.at[didx.at[j]], add=True)
        return 0
      lax.fori_loop(0, GS, body, 0)
      return 0
    lax.fori_loop(0, n_groups, group, 0)
    plsc.subcore_barrier()

    def readout(z, _):
      pltpu.async_copy(acc.at[myrows.at[z]], zb, sem).wait()
      pltpu.sync_copy(zb, _out_rows(out, cid, row0, z))
      return 0
    lax.fori_loop(0, n_zchunks, readout, 0)

  return mp_kernel


def _norms_from_hist(hist_blk):
  deg = hist_blk[0] + hist_blk[1]
  norm_src = lax.rsqrt(jnp.maximum(deg[:, 0], 1.0))
  norm_dst = lax.rsqrt(jnp.maximum(deg[:, 64], 1.0))
  return norm_src, norm_dst


def _scale_matmul_body(hist_ref, x_ref, w_ref, o_ref):
  norm_src, _ = _norms_from_hist(hist_ref[...])
  o_ref[...] = (x_ref[...] * norm_src[:, None]) @ w_ref[...]


def _layer_mid_body(p_ref, hist_ref, b1_ref, w2_ref, o_ref):
  norm_src, norm_dst = _norms_from_hist(hist_ref[...])
  agg = p_ref[0] + p_ref[1]
  h = jnp.maximum(agg * norm_dst[:, None] + b1_ref[...], 0.0)
  o_ref[...] = (h * norm_src[:, None]) @ w2_ref[...]


def _layer_out_body(p_ref, hist_ref, b2_ref, o_ref):
  _, norm_dst = _norms_from_hist(hist_ref[...])
  agg = p_ref[0] + p_ref[1]
  o_ref[...] = agg * norm_dst[:, None] + b2_ref[...]


def kernel(in_feat, edge_index, W1, b1, W2, b2):
  N, D_IN = in_feat.shape
  E = edge_index.shape[1]
  D_H = W1.shape[1]
  D_OUT = W2.shape[1]

  NP = ((N + 1023) // 1024) * 1024
  B = 128
  GS = 16
  C = -(-E // (NW * B * GS)) * GS
  EP = NW * C * B

  trash = NP - 1
  src_p = jnp.full((EP,), trash, I32).at[:E].set(edge_index[0])
  dst_p = jnp.full((EP,), trash, I32).at[:E].set(edge_index[1])
  src3 = src_p.reshape(NW, C, B)
  dst3 = dst_p.reshape(NW, C, B)

  hist = _make_degree_kernel(NP, 2 * C, 64, 2 * GS)(
      src_p.reshape(NW, 2 * C, 64), dst_p.reshape(NW, 2 * C, 64))

  x_p = jnp.zeros((NP, D_IN), F32).at[:N].set(in_feat)

  RB = 1024
  n_blocks = NP // RB
  hist_spec = pl.BlockSpec((NC, RB, 128), lambda i: (0, i, 0))
  row_spec = pl.BlockSpec((RB, D_IN), lambda i: (i, 0))
  part_spec = pl.BlockSpec((NC, RB, D_H), lambda i: (0, i, 0))

  z1 = pl.pallas_call(
      _scale_matmul_body,
      grid=(n_blocks,),
      in_specs=[hist_spec, row_spec,
                pl.BlockSpec((D_IN, D_H), lambda i: (0, 0))],
      out_specs=pl.BlockSpec((RB, D_H), lambda i: (i, 0)),
      out_shape=jax.ShapeDtypeStruct((NP, D_H), F32),
  )(hist, x_p, W1)

  mp_kernel = _make_scatter_kernel(D_H, NP, C, B, GS)
  p1 = mp_kernel(z1, src3, dst3)

  z2 = pl.pallas_call(
      _layer_mid_body,
      grid=(n_blocks,),
      in_specs=[part_spec, hist_spec,
                pl.BlockSpec((1, D_H), lambda i: (0, 0)),
                pl.BlockSpec((D_H, D_OUT), lambda i: (0, 0))],
      out_specs=pl.BlockSpec((RB, D_OUT), lambda i: (i, 0)),
      out_shape=jax.ShapeDtypeStruct((NP, D_OUT), F32),
  )(p1, hist, b1.reshape(1, D_H), W2)

  z2 = z2.at[N:].set(0.0)

  p2 = mp_kernel(z2, src3, dst3)

  out = pl.pallas_call(
      _layer_out_body,
      grid=(n_blocks,),
      in_specs=[pl.BlockSpec((NC, RB, D_OUT), lambda i: (0, i, 0)), hist_spec,
                pl.BlockSpec((1, D_OUT), lambda i: (0, 0))],
      out_specs=pl.BlockSpec((RB, D_OUT), lambda i: (i, 0)),
      out_shape=jax.ShapeDtypeStruct((NP, D_OUT), F32),
  )(p2, hist, b2.reshape(1, D_OUT))

  return out[:N]


# speedup vs baseline: 6.6845x; 2.1656x over previous
import functools

import jax
import jax.numpy as jnp
from jax import lax
from jax.experimental import pallas as pl
from jax.experimental.pallas import tpu as pltpu
from jax.experimental.pallas import tpu_sc as plsc

NC = 2
NS = 16
NW = NC * NS

F32 = jnp.float32
I32 = jnp.int32

ZB = 32


def _fill_myrows(myrows, row0, n_chunks):
  ivec = lax.iota(I32, 16)
  one_i = jnp.full((16,), 1, I32)

  def fr(i, _):
    z = i // 2
    off = (i % 2) * 16
    myrows[z, pl.ds(off, 16)] = ivec + one_i * (row0 + i * 16)
    return 0
  lax.fori_loop(0, n_chunks * 2, fr, 0)


def _fill_zeros(zb, d_vecs):
  def fz(i, _):
    for c in range(d_vecs):
      zb[i, pl.ds(c * 16, 16)] = jnp.zeros((16,), F32)
    return 0
  lax.fori_loop(0, ZB, fz, 0)


def _out_rows(out, cid, row0, z):
  off = pl.multiple_of(row0 + z * ZB, ZB)
  return out.at[cid, pl.ds(off, ZB)]


def _make_degree_kernel(NP, C, B, GS):
  mesh = plsc.VectorSubcoreMesh(core_axis_name="c", subcore_axis_name="s",
                                num_cores=NC, num_subcores=NS)
  rpt = NP // NS
  n_zchunks = rpt // ZB
  n_groups = C // GS

  @functools.partial(
      pl.kernel,
      mesh=mesh,
      out_type=jax.ShapeDtypeStruct((NC, NP, 128), F32),
      scratch_types=[
          pltpu.VMEM_SHARED((NP, 128), F32),
          pltpu.VMEM((GS, B), I32),
          pltpu.VMEM((GS, B), I32),
          pltpu.VMEM((n_zchunks, ZB), I32),
          pltpu.VMEM((B, 128), F32),
          pltpu.VMEM((B, 128), F32),
          pltpu.VMEM((ZB, 128), F32),
          pltpu.SemaphoreType.DMA,
      ],
  )
  def deg_kernel(src3, dst3, out, acc, sidx, didx, myrows, ones_l, ones_r,
                 zb, sem):
    cid = lax.axis_index("c")
    tid = lax.axis_index("s")
    wid = cid * NS + tid
    row0 = tid * rpt

    def fill_ones(i, _):
      for c in range(8):
        v = 1.0 if c < 4 else 0.0
        ones_l[i, pl.ds(c * 16, 16)] = jnp.full((16,), v, F32)
        ones_r[i, pl.ds(c * 16, 16)] = jnp.full((16,), 1.0 - v, F32)
      return 0
    lax.fori_loop(0, B, fill_ones, 0)

    _fill_zeros(zb, 8)
    _fill_myrows(myrows, row0, n_zchunks)

    def zero(z, _):
      pltpu.sync_copy(zb, acc.at[myrows.at[z]])
      return 0
    lax.fori_loop(0, n_zchunks, zero, 0)
    plsc.subcore_barrier()

    def group(g, _):
      goff = pl.multiple_of(g * GS, GS)
      pltpu.async_copy(src3.at[wid, pl.ds(goff, GS)], sidx, sem).wait()
      pltpu.async_copy(dst3.at[wid, pl.ds(goff, GS)], didx, sem).wait()

      def body(j, _):
        pltpu.sync_copy(ones_l, acc.at[sidx.at[j]], add=True)
        pltpu.sync_copy(ones_r, acc.at[didx.at[j]], add=True)
        return 0
      lax.fori_loop(0, GS, body, 0)
      return 0
    lax.fori_loop(0, n_groups, group, 0)
    plsc.subcore_barrier()

    def readout(z, _):
      pltpu.async_copy(acc.at[myrows.at[z]], zb, sem).wait()
      pltpu.sync_copy(zb, _out_rows(out, cid, row0, z))
      return 0
    lax.fori_loop(0, n_zchunks, readout, 0)

  return deg_kernel


def _make_scatter_kernel(D, NP, C, B, GS):
  mesh = plsc.VectorSubcoreMesh(core_axis_name="c", subcore_axis_name="s",
                                num_cores=NC, num_subcores=NS)
  rpt = NP // NS
  n_zchunks = rpt // ZB
  n_groups = C // GS

  @functools.partial(
      pl.kernel,
      mesh=mesh,
      out_type=jax.ShapeDtypeStruct((NC, NP, D), F32),
      scratch_types=[
          pltpu.VMEM_SHARED((NP, D), F32),
          pltpu.VMEM((GS, B), I32),
          pltpu.VMEM((GS, B), I32),
          pltpu.VMEM((n_zchunks, ZB), I32),
          pltpu.VMEM((B, D), F32),
          pltpu.VMEM((ZB, D), F32),
          pltpu.SemaphoreType.DMA,
      ],
  )
  def mp_kernel(z_hbm, src3, dst3, out, acc, sidx, didx, myrows, rows_v, zb,
                sem):
    cid = lax.axis_index("c")
    tid = lax.axis_index("s")
    wid = cid * NS + tid
    row0 = tid * rpt

    _fill_zeros(zb, D // 16)
    _fill_myrows(myrows, row0, n_zchunks)

    def zero(z, _):
      pltpu.sync_copy(zb, acc.at[myrows.at[z]])
      return 0
    lax.fori_loop(0, n_zchunks, zero, 0)
    plsc.subcore_barrier()

    def group(g, _):
      goff = pl.multiple_of(g * GS, GS)
      pltpu.async_copy(src3.at[wid, pl.ds(goff, GS)], sidx, sem).wait()
      pltpu.async_copy(dst3.at[wid, pl.ds(goff, GS)], didx, sem).wait()

      def body(j, _):
        pltpu.async_copy(z_hbm.at[sidx.at[j]], rows_v, sem).wait()
        pltpu.sync_copy(rows_v, acc.at[didx.at[j]], add=True)
        return 0
      lax.fori_loop(0, GS, body, 0)
      return 0
    lax.fori_loop(0, n_groups, group, 0)
    plsc.subcore_barrier()

    def readout(z, _):
      pltpu.async_copy(acc.at[myrows.at[z]], zb, sem).wait()
      pltpu.sync_copy(zb, _out_rows(out, cid, row0, z))
      return 0
    lax.fori_loop(0, n_zchunks, readout, 0)

  return mp_kernel


def _norms_from_hist(hist_blk):
  deg = hist_blk[0] + hist_blk[1]
  norm_src = lax.rsqrt(jnp.maximum(deg[:, 0], 1.0))
  norm_dst = lax.rsqrt(jnp.maximum(deg[:, 64], 1.0))
  return norm_src, norm_dst


def _scale_matmul_body(hist_ref, x_ref, w_ref, o_ref):
  norm_src, _ = _norms_from_hist(hist_ref[...])
  o_ref[...] = (x_ref[...] * norm_src[:, None]) @ w_ref[...]


def _layer_mid_body(p_ref, hist_ref, b1_ref, w2_ref, o_ref):
  norm_src, norm_dst = _norms_from_hist(hist_ref[...])
  agg = p_ref[0] + p_ref[1]
  h = jnp.maximum(agg * norm_dst[:, None] + b1_ref[...], 0.0)
  o_ref[...] = (h * norm_src[:, None]) @ w2_ref[...]


def _layer_out_body(p_ref, hist_ref, b2_ref, o_ref):
  _, norm_dst = _norms_from_hist(hist_ref[...])
  agg = p_ref[0] + p_ref[1]
  o_ref[...] = agg * norm_dst[:, None] + b2_ref[...]


def kernel(in_feat, edge_index, W1, b1, W2, b2):
  N, D_IN = in_feat.shape
  E = edge_index.shape[1]
  D_H = W1.shape[1]
  D_OUT = W2.shape[1]

  NP = ((N + 1023) // 1024) * 1024
  B = 128
  GS = 16
  C = -(-E // (NW * B * GS)) * GS
  EP = NW * C * B

  spare = NP - N
  trash_rows = (N + jnp.arange(EP, dtype=I32) % spare).astype(I32)
  src_p = trash_rows.at[:E].set(edge_index[0])
  dst_p = trash_rows.at[:E].set(edge_index[1])
  src3 = src_p.reshape(NW, C, B)
  dst3 = dst_p.reshape(NW, C, B)

  hist = _make_degree_kernel(NP, 2 * C, 64, 2 * GS)(
      src_p.reshape(NW, 2 * C, 64), dst_p.reshape(NW, 2 * C, 64))

  x_p = jnp.zeros((NP, D_IN), F32).at[:N].set(in_feat)

  RB = 1024
  n_blocks = NP // RB
  hist_spec = pl.BlockSpec((NC, RB, 128), lambda i: (0, i, 0))
  row_spec = pl.BlockSpec((RB, D_IN), lambda i: (i, 0))
  part_spec = pl.BlockSpec((NC, RB, D_H), lambda i: (0, i, 0))

  z1 = pl.pallas_call(
      _scale_matmul_body,
      grid=(n_blocks,),
      in_specs=[hist_spec, row_spec,
                pl.BlockSpec((D_IN, D_H), lambda i: (0, 0))],
      out_specs=pl.BlockSpec((RB, D_H), lambda i: (i, 0)),
      out_shape=jax.ShapeDtypeStruct((NP, D_H), F32),
  )(hist, x_p, W1)

  mp_kernel = _make_scatter_kernel(D_H, NP, C, B, GS)
  p1 = mp_kernel(z1, src3, dst3)

  z2 = pl.pallas_call(
      _layer_mid_body,
      grid=(n_blocks,),
      in_specs=[part_spec, hist_spec,
                pl.BlockSpec((1, D_H), lambda i: (0, 0)),
                pl.BlockSpec((D_H, D_OUT), lambda i: (0, 0))],
      out_specs=pl.BlockSpec((RB, D_OUT), lambda i: (i, 0)),
      out_shape=jax.ShapeDtypeStruct((NP, D_OUT), F32),
  )(p1, hist, b1.reshape(1, D_H), W2)

  z2 = z2.at[N:].set(0.0)

  p2 = mp_kernel(z2, src3, dst3)

  out = pl.pallas_call(
      _layer_out_body,
      grid=(n_blocks,),
      in_specs=[pl.BlockSpec((NC, RB, D_OUT), lambda i: (0, i, 0)), hist_spec,
                pl.BlockSpec((1, D_OUT), lambda i: (0, 0))],
      out_specs=pl.BlockSpec((RB, D_OUT), lambda i: (i, 0)),
      out_shape=jax.ShapeDtypeStruct((NP, D_OUT), F32),
  )(p2, hist, b2.reshape(1, D_OUT))

  return out[:N]


# speedup vs baseline: 8.5755x; 1.2829x over previous
import functools

import jax
import jax.numpy as jnp
from jax import lax
from jax.experimental import pallas as pl
from jax.experimental.pallas import tpu as pltpu
from jax.experimental.pallas import tpu_sc as plsc

NC = 2
NS = 16
NW = NC * NS

F32 = jnp.float32
I32 = jnp.int32

ZB = 32


def _fill_myrows(myrows, row0, n_chunks):
  ivec = lax.iota(I32, 16)
  one_i = jnp.full((16,), 1, I32)

  def fr(i, _):
    z = i // 2
    off = (i % 2) * 16
    myrows[z, pl.ds(off, 16)] = ivec + one_i * (row0 + i * 16)
    return 0
  lax.fori_loop(0, n_chunks * 2, fr, 0)


def _fill_zeros(zb, d_vecs):
  def fz(i, _):
    for c in range(d_vecs):
      zb[i, pl.ds(c * 16, 16)] = jnp.zeros((16,), F32)
    return 0
  lax.fori_loop(0, ZB, fz, 0)


def _out_rows(out, cid, row0, z):
  off = pl.multiple_of(row0 + z * ZB, ZB)
  return out.at[cid, pl.ds(off, ZB)]


def _make_degree_kernel(NP, C, B, GS):
  mesh = plsc.VectorSubcoreMesh(core_axis_name="c", subcore_axis_name="s",
                                num_cores=NC, num_subcores=NS)
  rpt = NP // NS
  n_zchunks = rpt // ZB
  n_groups = C // GS

  @functools.partial(
      pl.kernel,
      mesh=mesh,
      out_type=jax.ShapeDtypeStruct((NC, NP, 128), F32),
      scratch_types=[
          pltpu.VMEM_SHARED((NP, 128), F32),
          pltpu.VMEM((GS, B), I32),
          pltpu.VMEM((GS, B), I32),
          pltpu.VMEM((n_zchunks, ZB), I32),
          pltpu.VMEM((B, 128), F32),
          pltpu.VMEM((B, 128), F32),
          pltpu.VMEM((ZB, 128), F32),
          pltpu.SemaphoreType.DMA,
          pltpu.SemaphoreType.DMA,
          pltpu.SemaphoreType.DMA,
          pltpu.SemaphoreType.DMA,
      ],
  )
  def deg_kernel(src3, dst3, out, acc, sidx, didx, myrows, ones_l, ones_r,
                 zb, sem, sem2, sem3, sem4):
    cid = lax.axis_index("c")
    tid = lax.axis_index("s")
    wid = cid * NS + tid
    row0 = tid * rpt

    def fill_ones(i, _):
      for c in range(8):
        v = 1.0 if c < 4 else 0.0
        ones_l[i, pl.ds(c * 16, 16)] = jnp.full((16,), v, F32)
        ones_r[i, pl.ds(c * 16, 16)] = jnp.full((16,), 1.0 - v, F32)
      return 0
    lax.fori_loop(0, B, fill_ones, 0)

    _fill_zeros(zb, 8)
    _fill_myrows(myrows, row0, n_zchunks)

    def zero(z, _):
      pltpu.sync_copy(zb, acc.at[myrows.at[z]])
      return 0
    lax.fori_loop(0, n_zchunks, zero, 0)
    plsc.subcore_barrier()

    def group(g, _):
      goff = pl.multiple_of(g * GS, GS)
      pltpu.async_copy(src3.at[wid, pl.ds(goff, GS)], sidx, sem).wait()
      pltpu.async_copy(dst3.at[wid, pl.ds(goff, GS)], didx, sem).wait()

      def body(t, _):
        j0 = 2 * t
        j1 = 2 * t + 1
        a0 = pltpu.async_copy(ones_l, acc.at[sidx.at[j0]], sem, add=True)
        a1 = pltpu.async_copy(ones_r, acc.at[didx.at[j0]], sem2, add=True)
        a2 = pltpu.async_copy(ones_l, acc.at[sidx.at[j1]], sem3, add=True)
        a3 = pltpu.async_copy(ones_r, acc.at[didx.at[j1]], sem4, add=True)
        a0.wait()
        a1.wait()
        a2.wait()
        a3.wait()
        return 0
      lax.fori_loop(0, GS // 2, body, 0)
      return 0
    lax.fori_loop(0, n_groups, group, 0)
    plsc.subcore_barrier()

    def readout(z, _):
      pltpu.async_copy(acc.at[myrows.at[z]], zb, sem).wait()
      pltpu.sync_copy(zb, _out_rows(out, cid, row0, z))
      return 0
    lax.fori_loop(0, n_zchunks, readout, 0)

  return deg_kernel


def _make_scatter_kernel(D, NP, C, B, GS):
  mesh = plsc.VectorSubcoreMesh(core_axis_name="c", subcore_axis_name="s",
                                num_cores=NC, num_subcores=NS)
  rpt = NP // NS
  n_zchunks = rpt // ZB
  n_groups = C // GS

  @functools.partial(
      pl.kernel,
      mesh=mesh,
      out_type=jax.ShapeDtypeStruct((NC, NP, D), F32),
      scratch_types=[
          pltpu.VMEM_SHARED((NP, D), F32),
          pltpu.VMEM((GS, B), I32),
          pltpu.VMEM((GS, B), I32),
          pltpu.VMEM((n_zchunks, ZB), I32),
          pltpu.VMEM((B, D), F32),
          pltpu.VMEM((B, D), F32),
          pltpu.VMEM((ZB, D), F32),
          pltpu.SemaphoreType.DMA,
          pltpu.SemaphoreType.DMA,
          pltpu.SemaphoreType.DMA,
      ],
  )
  def mp_kernel(z_hbm, src3, dst3, out, acc, sidx, didx, myrows, rows_a,
                rows_b, zb, sem, sem_a, sem_b):
    cid = lax.axis_index("c")
    tid = lax.axis_index("s")
    wid = cid * NS + tid
    row0 = tid * rpt

    _fill_zeros(zb, D // 16)
    _fill_myrows(myrows, row0, n_zchunks)

    def zero(z, _):
      pltpu.sync_copy(zb, acc.at[myrows.at[z]])
      return 0
    lax.fori_loop(0, n_zchunks, zero, 0)
    plsc.subcore_barrier()

    def group(g, _):
      goff = pl.multiple_of(g * GS, GS)
      pltpu.async_copy(src3.at[wid, pl.ds(goff, GS)], sidx, sem).wait()
      pltpu.async_copy(dst3.at[wid, pl.ds(goff, GS)], didx, sem).wait()

      pltpu.async_copy(z_hbm.at[sidx.at[0]], rows_a, sem_a)

      def pair(t, _):
        j0 = 2 * t
        j1 = 2 * t + 1
        gb = pltpu.async_copy(z_hbm.at[sidx.at[j1]], rows_b, sem_b)
        pltpu.make_async_copy(z_hbm.at[sidx.at[j0]], rows_a, sem_a).wait()
        pltpu.sync_copy(rows_a, acc.at[didx.at[j0]], add=True)

        @pl.when(t < GS // 2 - 1)
        def _():
          pltpu.async_copy(z_hbm.at[sidx.at[j0 + 2]], rows_a, sem_a)

        gb.wait()
        pltpu.sync_copy(rows_b, acc.at[didx.at[j1]], add=True)
        return 0
      lax.fori_loop(0, GS // 2, pair, 0)
      return 0
    lax.fori_loop(0, n_groups, group, 0)
    plsc.subcore_barrier()

    def readout(z, _):
      pltpu.async_copy(acc.at[myrows.at[z]], zb, sem).wait()
      pltpu.sync_copy(zb, _out_rows(out, cid, row0, z))
      return 0
    lax.fori_loop(0, n_zchunks, readout, 0)

  return mp_kernel


def _norms_from_hist(hist_blk):
  deg = hist_blk[0] + hist_blk[1]
  norm_src = lax.rsqrt(jnp.maximum(deg[:, 0], 1.0))
  norm_dst = lax.rsqrt(jnp.maximum(deg[:, 64], 1.0))
  return norm_src, norm_dst


def _scale_matmul_body(hist_ref, x_ref, w_ref, o_ref):
  norm_src, _ = _norms_from_hist(hist_ref[...])
  o_ref[...] = (x_ref[...] * norm_src[:, None]) @ w_ref[...]


def _layer_mid_body(p_ref, hist_ref, b1_ref, w2_ref, o_ref):
  norm_src, norm_dst = _norms_from_hist(hist_ref[...])
  agg = p_ref[0] + p_ref[1]
  h = jnp.maximum(agg * norm_dst[:, None] + b1_ref[...], 0.0)
  o_ref[...] = (h * norm_src[:, None]) @ w2_ref[...]


def _layer_out_body(p_ref, hist_ref, b2_ref, o_ref):
  _, norm_dst = _norms_from_hist(hist_ref[...])
  agg = p_ref[0] + p_ref[1]
  o_ref[...] = agg * norm_dst[:, None] + b2_ref[...]


def kernel(in_feat, edge_index, W1, b1, W2, b2):
  N, D_IN = in_feat.shape
  E = edge_index.shape[1]
  D_H = W1.shape[1]
  D_OUT = W2.shape[1]

  NP = ((N + 1023) // 1024) * 1024
  B = 128
  GS = 16
  C = -(-E // (NW * B * GS)) * GS
  EP = NW * C * B

  spare = NP - N
  trash_rows = (N + jnp.arange(EP, dtype=I32) % spare).astype(I32)
  src_p = trash_rows.at[:E].set(edge_index[0])
  dst_p = trash_rows.at[:E].set(edge_index[1])
  src3 = src_p.reshape(NW, C, B)
  dst3 = dst_p.reshape(NW, C, B)

  hist = _make_degree_kernel(NP, 2 * C, 64, 2 * GS)(
      src_p.reshape(NW, 2 * C, 64), dst_p.reshape(NW, 2 * C, 64))

  x_p = jnp.zeros((NP, D_IN), F32).at[:N].set(in_feat)

  RB = 1024
  n_blocks = NP // RB
  hist_spec = pl.BlockSpec((NC, RB, 128), lambda i: (0, i, 0))
  row_spec = pl.BlockSpec((RB, D_IN), lambda i: (i, 0))
  part_spec = pl.BlockSpec((NC, RB, D_H), lambda i: (0, i, 0))

  z1 = pl.pallas_call(
      _scale_matmul_body,
      grid=(n_blocks,),
      in_specs=[hist_spec, row_spec,
                pl.BlockSpec((D_IN, D_H), lambda i: (0, 0))],
      out_specs=pl.BlockSpec((RB, D_H), lambda i: (i, 0)),
      out_shape=jax.ShapeDtypeStruct((NP, D_H), F32),
  )(hist, x_p, W1)

  mp_kernel = _make_scatter_kernel(D_H, NP, C, B, GS)
  p1 = mp_kernel(z1, src3, dst3)

  z2 = pl.pallas_call(
      _layer_mid_body,
      grid=(n_blocks,),
      in_specs=[part_spec, hist_spec,
                pl.BlockSpec((1, D_H), lambda i: (0, 0)),
                pl.BlockSpec((D_H, D_OUT), lambda i: (0, 0))],
      out_specs=pl.BlockSpec((RB, D_OUT), lambda i: (i, 0)),
      out_shape=jax.ShapeDtypeStruct((NP, D_OUT), F32),
  )(p1, hist, b1.reshape(1, D_H), W2)

  z2 = z2.at[N:].set(0.0)

  p2 = mp_kernel(z2, src3, dst3)

  out = pl.pallas_call(
      _layer_out_body,
      grid=(n_blocks,),
      in_specs=[pl.BlockSpec((NC, RB, D_OUT), lambda i: (0, i, 0)), hist_spec,
                pl.BlockSpec((1, D_OUT), lambda i: (0, 0))],
      out_specs=pl.BlockSpec((RB, D_OUT), lambda i: (i, 0)),
      out_shape=jax.ShapeDtypeStruct((NP, D_OUT), F32),
  )(p2, hist, b2.reshape(1, D_OUT))

  return out[:N]


# speedup vs baseline: 8.7601x; 1.0215x over previous
import functools

import jax
import jax.numpy as jnp
from jax import lax
from jax.experimental import pallas as pl
from jax.experimental.pallas import tpu as pltpu
from jax.experimental.pallas import tpu_sc as plsc

NC = 2
NS = 16
NW = NC * NS

F32 = jnp.float32
I32 = jnp.int32

def _fill_myrows(myrows, row0, n_chunks, zbw):
  spr = zbw // 16
  ivec = lax.iota(I32, 16)
  one_i = jnp.full((16,), 1, I32)

  def fr(i, _):
    z = i // spr
    off = (i % spr) * 16
    myrows[z, pl.ds(off, 16)] = ivec + one_i * (row0 + i * 16)
    return 0
  lax.fori_loop(0, n_chunks * spr, fr, 0)


def _fill_zeros(zb, n_rows, d_vecs):
  def fz(i, _):
    for c in range(d_vecs):
      zb[i, pl.ds(c * 16, 16)] = jnp.zeros((16,), F32)
    return 0
  lax.fori_loop(0, n_rows, fz, 0)


def _out_rows(out, cid, row0, z, zbw):
  off = pl.multiple_of(row0 + z * zbw, zbw)
  return out.at[cid, pl.ds(off, zbw)]


def _make_degree_kernel(NP, C, B, GS):
  mesh = plsc.VectorSubcoreMesh(core_axis_name="c", subcore_axis_name="s",
                                num_cores=NC, num_subcores=NS)
  rpt = NP // NS
  ZBW = 64
  n_zchunks = rpt // ZBW
  n_groups = C // GS

  @functools.partial(
      pl.kernel,
      mesh=mesh,
      out_type=jax.ShapeDtypeStruct((NC, NP, 128), F32),
      scratch_types=[
          pltpu.VMEM_SHARED((NP, 128), F32),
          pltpu.VMEM((GS, B), I32),
          pltpu.VMEM((GS, B), I32),
          pltpu.VMEM((n_zchunks, ZBW), I32),
          pltpu.VMEM((B, 128), F32),
          pltpu.VMEM((B, 128), F32),
          pltpu.VMEM((ZBW, 128), F32),
          pltpu.SemaphoreType.DMA,
          pltpu.SemaphoreType.DMA,
          pltpu.SemaphoreType.DMA,
          pltpu.SemaphoreType.DMA,
      ],
  )
  def deg_kernel(src3, dst3, out, acc, sidx, didx, myrows, ones_l, ones_r,
                 zb, sem, sem2, sem3, sem4):
    cid = lax.axis_index("c")
    tid = lax.axis_index("s")
    wid = cid * NS + tid
    row0 = tid * rpt

    def fill_ones(i, _):
      for c in range(8):
        v = 1.0 if c < 4 else 0.0
        ones_l[i, pl.ds(c * 16, 16)] = jnp.full((16,), v, F32)
        ones_r[i, pl.ds(c * 16, 16)] = jnp.full((16,), 1.0 - v, F32)
      return 0
    lax.fori_loop(0, B, fill_ones, 0)

    _fill_zeros(zb, ZBW, 8)
    _fill_myrows(myrows, row0, n_zchunks, ZBW)

    def zero(z, _):
      pltpu.sync_copy(zb, acc.at[myrows.at[z]])
      return 0
    lax.fori_loop(0, n_zchunks, zero, 0)
    plsc.subcore_barrier()

    def group(g, _):
      goff = pl.multiple_of(g * GS, GS)
      pltpu.async_copy(src3.at[wid, pl.ds(goff, GS)], sidx, sem).wait()
      pltpu.async_copy(dst3.at[wid, pl.ds(goff, GS)], didx, sem).wait()

      def body(t, _):
        j0 = 2 * t
        j1 = 2 * t + 1
        a0 = pltpu.async_copy(ones_l, acc.at[sidx.at[j0]], sem, add=True)
        a1 = pltpu.async_copy(ones_r, acc.at[didx.at[j0]], sem2, add=True)
        a2 = pltpu.async_copy(ones_l, acc.at[sidx.at[j1]], sem3, add=True)
        a3 = pltpu.async_copy(ones_r, acc.at[didx.at[j1]], sem4, add=True)
        a0.wait()
        a1.wait()
        a2.wait()
        a3.wait()
        return 0
      lax.fori_loop(0, GS // 2, body, 0)
      return 0
    lax.fori_loop(0, n_groups, group, 0)
    plsc.subcore_barrier()

    def readout(z, _):
      pltpu.async_copy(acc.at[myrows.at[z]], zb, sem).wait()
      pltpu.sync_copy(zb, _out_rows(out, cid, row0, z, ZBW))
      return 0
    lax.fori_loop(0, n_zchunks, readout, 0)

  return deg_kernel


def _make_scatter_kernel(D, NP, C, B, GS):
  mesh = plsc.VectorSubcoreMesh(core_axis_name="c", subcore_axis_name="s",
                                num_cores=NC, num_subcores=NS)
  rpt = NP // NS
  ZBW = 128
  n_zchunks = rpt // ZBW
  n_groups = C // GS

  @functools.partial(
      pl.kernel,
      mesh=mesh,
      out_type=jax.ShapeDtypeStruct((NC, NP, D), F32),
      scratch_types=[
          pltpu.VMEM_SHARED((NP, D), F32),
          pltpu.VMEM((GS, B), I32),
          pltpu.VMEM((GS, B), I32),
          pltpu.VMEM((n_zchunks, ZBW), I32),
          pltpu.VMEM((B, D), F32),
          pltpu.VMEM((B, D), F32),
          pltpu.SemaphoreType.DMA,
          pltpu.SemaphoreType.DMA,
          pltpu.SemaphoreType.DMA,
      ],
  )
  def mp_kernel(z_hbm, src3, dst3, out, acc, sidx, didx, myrows, rows_a,
                rows_b, sem, sem_a, sem_b):
    cid = lax.axis_index("c")
    tid = lax.axis_index("s")
    wid = cid * NS + tid
    row0 = tid * rpt

    _fill_zeros(rows_a, ZBW, D // 16)
    _fill_myrows(myrows, row0, n_zchunks, ZBW)

    def zero(z, _):
      pltpu.sync_copy(rows_a, acc.at[myrows.at[z]])
      return 0
    lax.fori_loop(0, n_zchunks, zero, 0)
    plsc.subcore_barrier()

    def group(g, _):
      goff = pl.multiple_of(g * GS, GS)
      pltpu.async_copy(src3.at[wid, pl.ds(goff, GS)], sidx, sem).wait()
      pltpu.async_copy(dst3.at[wid, pl.ds(goff, GS)], didx, sem).wait()

      pltpu.async_copy(z_hbm.at[sidx.at[0]], rows_a, sem_a)

      def pair(t, _):
        j0 = 2 * t
        j1 = 2 * t + 1
        gb = pltpu.async_copy(z_hbm.at[sidx.at[j1]], rows_b, sem_b)
        pltpu.make_async_copy(z_hbm.at[sidx.at[j0]], rows_a, sem_a).wait()
        pltpu.sync_copy(rows_a, acc.at[didx.at[j0]], add=True)

        @pl.when(t < GS // 2 - 1)
        def _():
          pltpu.async_copy(z_hbm.at[sidx.at[j0 + 2]], rows_a, sem_a)

        gb.wait()
        pltpu.sync_copy(rows_b, acc.at[didx.at[j1]], add=True)
        return 0
      lax.fori_loop(0, GS // 2, pair, 0)
      return 0
    lax.fori_loop(0, n_groups, group, 0)
    plsc.subcore_barrier()

    def readout(z, _):
      pltpu.async_copy(acc.at[myrows.at[z]], rows_a, sem).wait()
      pltpu.sync_copy(rows_a, _out_rows(out, cid, row0, z, ZBW))
      return 0
    lax.fori_loop(0, n_zchunks, readout, 0)

  return mp_kernel


def _norms_from_hist(hist_blk):
  deg = hist_blk[0] + hist_blk[1]
  norm_src = lax.rsqrt(jnp.maximum(deg[:, 0], 1.0))
  norm_dst = lax.rsqrt(jnp.maximum(deg[:, 64], 1.0))
  return norm_src, norm_dst


def _matmul_body(x_ref, w_ref, o_ref):
  o_ref[...] = x_ref[...] @ w_ref[...]


def _scale_body(hist_ref, y_ref, o_ref):
  norm_src, _ = _norms_from_hist(hist_ref[...])
  o_ref[...] = y_ref[...] * norm_src[:, None]


def _make_layer_mid_body(N, RB):
  def _layer_mid_body(p_ref, hist_ref, b1_ref, w2_ref, o_ref):
    norm_src, norm_dst = _norms_from_hist(hist_ref[...])
    agg = p_ref[0] + p_ref[1]
    h = jnp.maximum(agg * norm_dst[:, None] + b1_ref[...], 0.0)
    z = (h * norm_src[:, None]) @ w2_ref[...]
    row = pl.program_id(0) * RB + lax.broadcasted_iota(I32, (RB, 1), 0)
    o_ref[...] = jnp.where(row < N, z, 0.0)
  return _layer_mid_body


def _layer_out_body(p_ref, hist_ref, b2_ref, o_ref):
  _, norm_dst = _norms_from_hist(hist_ref[...])
  agg = p_ref[0] + p_ref[1]
  o_ref[...] = agg * norm_dst[:, None] + b2_ref[...]


def kernel(in_feat, edge_index, W1, b1, W2, b2):
  N, D_IN = in_feat.shape
  E = edge_index.shape[1]
  D_H = W1.shape[1]
  D_OUT = W2.shape[1]

  NP = ((N + 1023) // 1024) * 1024
  B = 128
  GS = 16
  C = -(-E // (NW * B * GS)) * GS
  EP = NW * C * B

  spare = NP - N
  trash_rows = (N + jnp.arange(EP, dtype=I32) % spare).astype(I32)
  src_p = trash_rows.at[:E].set(edge_index[0])
  dst_p = trash_rows.at[:E].set(edge_index[1])
  src3 = src_p.reshape(NW, C, B)
  dst3 = dst_p.reshape(NW, C, B)

  x_p = jnp.zeros((NP, D_IN), F32).at[:N].set(in_feat)

  RB = 1024
  n_blocks = NP // RB
  hist_spec = pl.BlockSpec((NC, RB, 128), lambda i: (0, i, 0))
  row_spec = pl.BlockSpec((RB, D_IN), lambda i: (i, 0))
  part_spec = pl.BlockSpec((NC, RB, D_H), lambda i: (0, i, 0))

  y1 = pl.pallas_call(
      _matmul_body,
      grid=(n_blocks,),
      in_specs=[row_spec, pl.BlockSpec((D_IN, D_H), lambda i: (0, 0))],
      out_specs=pl.BlockSpec((RB, D_H), lambda i: (i, 0)),
      out_shape=jax.ShapeDtypeStruct((NP, D_H), F32),
  )(x_p, W1)

  hist = _make_degree_kernel(NP, 2 * C, 64, 2 * GS)(
      src_p.reshape(NW, 2 * C, 64), dst_p.reshape(NW, 2 * C, 64))

  z1 = pl.pallas_call(
      _scale_body,
      grid=(n_blocks,),
      in_specs=[hist_spec, pl.BlockSpec((RB, D_H), lambda i: (i, 0))],
      out_specs=pl.BlockSpec((RB, D_H), lambda i: (i, 0)),
      out_shape=jax.ShapeDtypeStruct((NP, D_H), F32),
  )(hist, y1)

  mp_kernel = _make_scatter_kernel(D_H, NP, C, B, GS)
  p1 = mp_kernel(z1, src3, dst3)

  z2 = pl.pallas_call(
      _make_layer_mid_body(N, RB),
      grid=(n_blocks,),
      in_specs=[part_spec, hist_spec,
                pl.BlockSpec((1, D_H), lambda i: (0, 0)),
                pl.BlockSpec((D_H, D_OUT), lambda i: (0, 0))],
      out_specs=pl.BlockSpec((RB, D_OUT), lambda i: (i, 0)),
      out_shape=jax.ShapeDtypeStruct((NP, D_OUT), F32),
  )(p1, hist, b1.reshape(1, D_H), W2)

  p2 = mp_kernel(z2, src3, dst3)

  out = pl.pallas_call(
      _layer_out_body,
      grid=(n_blocks,),
      in_specs=[pl.BlockSpec((NC, RB, D_OUT), lambda i: (0, i, 0)), hist_spec,
                pl.BlockSpec((1, D_OUT), lambda i: (0, 0))],
      out_specs=pl.BlockSpec((RB, D_OUT), lambda i: (i, 0)),
      out_shape=jax.ShapeDtypeStruct((NP, D_OUT), F32),
  )(p2, hist, b2.reshape(1, D_OUT))

  return out[:N]


# speedup vs baseline: 9.0931x; 1.0380x over previous
import functools

import jax
import jax.numpy as jnp
from jax import lax
from jax.experimental import pallas as pl
from jax.experimental.pallas import tpu as pltpu
from jax.experimental.pallas import tpu_sc as plsc

NC = 2
NS = 16
NW = NC * NS

F32 = jnp.float32
I32 = jnp.int32

def _fill_myrows(myrows, row0, n_chunks, zbw):
  spr = zbw // 16
  ivec = lax.iota(I32, 16)
  one_i = jnp.full((16,), 1, I32)

  def fr(i, _):
    z = i // spr
    off = (i % spr) * 16
    myrows[z, pl.ds(off, 16)] = ivec + one_i * (row0 + i * 16)
    return 0
  lax.fori_loop(0, n_chunks * spr, fr, 0)


def _fill_zeros(zb, n_rows, d_vecs):
  def fz(i, _):
    for c in range(d_vecs):
      zb[i, pl.ds(c * 16, 16)] = jnp.zeros((16,), F32)
    return 0
  lax.fori_loop(0, n_rows, fz, 0)


def _out_rows(out, cid, row0, z, zbw):
  off = pl.multiple_of(row0 + z * zbw, zbw)
  return out.at[cid, pl.ds(off, zbw)]


def _make_degree_kernel(NP, C, B, GS):
  mesh = plsc.VectorSubcoreMesh(core_axis_name="c", subcore_axis_name="s",
                                num_cores=NC, num_subcores=NS)
  rpt = NP // NS
  ZBW = 64
  n_zchunks = rpt // ZBW
  n_groups = C // GS

  @functools.partial(
      pl.kernel,
      mesh=mesh,
      out_type=jax.ShapeDtypeStruct((NC, NP, 128), F32),
      scratch_types=[
          pltpu.VMEM_SHARED((NP, 128), F32),
          pltpu.VMEM((GS, B), I32),
          pltpu.VMEM((GS, B), I32),
          pltpu.VMEM((n_zchunks, ZBW), I32),
          pltpu.VMEM((B, 128), F32),
          pltpu.VMEM((B, 128), F32),
          pltpu.VMEM((ZBW, 128), F32),
          pltpu.SemaphoreType.DMA,
          pltpu.SemaphoreType.DMA,
          pltpu.SemaphoreType.DMA,
          pltpu.SemaphoreType.DMA,
      ],
  )
  def deg_kernel(src3, dst3, out, acc, sidx, didx, myrows, ones_l, ones_r,
                 zb, sem, sem2, sem3, sem4):
    cid = lax.axis_index("c")
    tid = lax.axis_index("s")
    wid = cid * NS + tid
    row0 = tid * rpt

    def fill_ones(i, _):
      for c in range(8):
        v = 1.0 if c < 4 else 0.0
        ones_l[i, pl.ds(c * 16, 16)] = jnp.full((16,), v, F32)
        ones_r[i, pl.ds(c * 16, 16)] = jnp.full((16,), 1.0 - v, F32)
      return 0
    lax.fori_loop(0, B, fill_ones, 0)

    _fill_zeros(zb, ZBW, 8)
    _fill_myrows(myrows, row0, n_zchunks, ZBW)

    def zero(z, _):
      pltpu.sync_copy(zb, acc.at[myrows.at[z]])
      return 0
    lax.fori_loop(0, n_zchunks, zero, 0)
    plsc.subcore_barrier()

    def group(g, _):
      goff = pl.multiple_of(g * GS, GS)
      pltpu.async_copy(src3.at[wid, pl.ds(goff, GS)], sidx, sem).wait()
      pltpu.async_copy(dst3.at[wid, pl.ds(goff, GS)], didx, sem).wait()

      def body(t, _):
        j0 = 2 * t
        j1 = 2 * t + 1
        a0 = pltpu.async_copy(ones_l, acc.at[sidx.at[j0]], sem, add=True)
        a1 = pltpu.async_copy(ones_r, acc.at[didx.at[j0]], sem2, add=True)
        a2 = pltpu.async_copy(ones_l, acc.at[sidx.at[j1]], sem3, add=True)
        a3 = pltpu.async_copy(ones_r, acc.at[didx.at[j1]], sem4, add=True)
        a0.wait()
        a1.wait()
        a2.wait()
        a3.wait()
        return 0
      lax.fori_loop(0, GS // 2, body, 0)
      return 0
    lax.fori_loop(0, n_groups, group, 0)
    plsc.subcore_barrier()

    def readout(z, _):
      pltpu.async_copy(acc.at[myrows.at[z]], zb, sem).wait()
      pltpu.sync_copy(zb, _out_rows(out, cid, row0, z, ZBW))
      return 0
    lax.fori_loop(0, n_zchunks, readout, 0)

  return deg_kernel


def _make_scatter_kernel(D, NP, C, B, GS):
  mesh = plsc.VectorSubcoreMesh(core_axis_name="c", subcore_axis_name="s",
                                num_cores=NC, num_subcores=NS)
  rpt = NP // NS
  ZBW = 128
  n_zchunks = rpt // ZBW
  n_groups = C // GS

  @functools.partial(
      pl.kernel,
      mesh=mesh,
      out_type=jax.ShapeDtypeStruct((NC, NP, D), F32),
      scratch_types=[
          pltpu.VMEM_SHARED((NP, D), F32),
          pltpu.VMEM((GS, B), I32),
          pltpu.VMEM((GS, B), I32),
          pltpu.VMEM((n_zchunks, ZBW), I32),
          pltpu.VMEM((B, D), F32),
          pltpu.VMEM((B, D), F32),
          pltpu.SemaphoreType.DMA,
          pltpu.SemaphoreType.DMA,
          pltpu.SemaphoreType.DMA,
      ],
  )
  def mp_kernel(z_hbm, src3, dst3, out, acc, sidx, didx, myrows, rows_a,
                rows_b, sem, sem_a, sem_b):
    cid = lax.axis_index("c")
    tid = lax.axis_index("s")
    wid = cid * NS + tid
    row0 = tid * rpt

    _fill_zeros(rows_a, ZBW, D // 16)
    _fill_myrows(myrows, row0, n_zchunks, ZBW)

    def zero(z, _):
      pltpu.sync_copy(rows_a, acc.at[myrows.at[z]])
      return 0
    lax.fori_loop(0, n_zchunks, zero, 0)
    plsc.subcore_barrier()

    def group(g, _):
      goff = pl.multiple_of(g * GS, GS)
      pltpu.async_copy(src3.at[wid, pl.ds(goff, GS)], sidx, sem).wait()
      pltpu.async_copy(dst3.at[wid, pl.ds(goff, GS)], didx, sem).wait()

      pltpu.async_copy(z_hbm.at[sidx.at[0]], rows_a, sem_a)

      def pair(t, _):
        j0 = 2 * t
        j1 = 2 * t + 1
        gb = pltpu.async_copy(z_hbm.at[sidx.at[j1]], rows_b, sem_b)
        pltpu.make_async_copy(z_hbm.at[sidx.at[j0]], rows_a, sem_a).wait()
        pltpu.sync_copy(rows_a, acc.at[didx.at[j0]], add=True)

        @pl.when(t < GS // 2 - 1)
        def _():
          pltpu.async_copy(z_hbm.at[sidx.at[j0 + 2]], rows_a, sem_a)

        gb.wait()
        pltpu.sync_copy(rows_b, acc.at[didx.at[j1]], add=True)
        return 0
      lax.fori_loop(0, GS // 2, pair, 0)
      return 0
    lax.fori_loop(0, n_groups, group, 0)
    plsc.subcore_barrier()

    def readout(z, _):
      pltpu.async_copy(acc.at[myrows.at[z]], rows_a, sem).wait()
      pltpu.sync_copy(rows_a, _out_rows(out, cid, row0, z, ZBW))
      return 0
    lax.fori_loop(0, n_zchunks, readout, 0)

  return mp_kernel


def _norms_from_hist(hist_blk):
  deg = hist_blk[0] + hist_blk[1]
  norm_src = lax.rsqrt(jnp.maximum(deg[:, 0], 1.0))
  norm_dst = lax.rsqrt(jnp.maximum(deg[:, 64], 1.0))
  return norm_src, norm_dst


def _matmul_body(x_ref, w_ref, o_ref):
  o_ref[...] = x_ref[...] @ w_ref[...]


def _scale_body(hist_ref, y_ref, o_ref):
  norm_src, _ = _norms_from_hist(hist_ref[...])
  o_ref[...] = y_ref[...] * norm_src[:, None]


def _make_layer_mid_body(N, RB):
  def _layer_mid_body(p_ref, hist_ref, b1_ref, w2_ref, o_ref):
    norm_src, norm_dst = _norms_from_hist(hist_ref[...])
    agg = p_ref[0] + p_ref[1]
    h = jnp.maximum(agg * norm_dst[:, None] + b1_ref[...], 0.0)
    z = (h * norm_src[:, None]) @ w2_ref[...]
    row = pl.program_id(0) * RB + lax.broadcasted_iota(I32, (RB, 1), 0)
    o_ref[...] = jnp.where(row < N, z, 0.0)
  return _layer_mid_body


def _layer_out_body(p_ref, hist_ref, b2_ref, o_ref):
  _, norm_dst = _norms_from_hist(hist_ref[...])
  agg = p_ref[0] + p_ref[1]
  o_ref[...] = agg * norm_dst[:, None] + b2_ref[...]


def kernel(in_feat, edge_index, W1, b1, W2, b2):
  N, D_IN = in_feat.shape
  E = edge_index.shape[1]
  D_H = W1.shape[1]
  D_OUT = W2.shape[1]

  NP = ((N + 1023) // 1024) * 1024
  B = 128
  GS = 16
  C = -(-E // (NW * B * GS)) * GS
  EP = NW * C * B

  spare = NP - N
  trash_rows = (N + jnp.arange(EP, dtype=I32) % spare).astype(I32)
  src_p = trash_rows.at[:E].set(edge_index[0])
  dst_p = trash_rows.at[:E].set(edge_index[1])
  src3 = src_p.reshape(NW, C, B)
  dst3 = dst_p.reshape(NW, C, B)

  x_p = jnp.zeros((NP, D_IN), F32).at[:N].set(in_feat)

  RB = 1024
  n_blocks = NP // RB
  hist_spec = pl.BlockSpec((NC, RB, 128), lambda i: (0, i, 0))
  row_spec = pl.BlockSpec((RB, D_IN), lambda i: (i, 0))
  part_spec = pl.BlockSpec((NC, RB, D_H), lambda i: (0, i, 0))

  y1 = pl.pallas_call(
      _matmul_body,
      grid=(n_blocks,),
      in_specs=[row_spec, pl.BlockSpec((D_IN, D_H), lambda i: (0, 0))],
      out_specs=pl.BlockSpec((RB, D_H), lambda i: (i, 0)),
      out_shape=jax.ShapeDtypeStruct((NP, D_H), F32),
  )(x_p, W1)

  hist = _make_degree_kernel(NP, 2 * C, 64, 2 * GS)(
      src_p.reshape(NW, 2 * C, 64), dst_p.reshape(NW, 2 * C, 64))

  z1 = pl.pallas_call(
      _scale_body,
      grid=(n_blocks,),
      in_specs=[hist_spec, pl.BlockSpec((RB, D_H), lambda i: (i, 0))],
      out_specs=pl.BlockSpec((RB, D_H), lambda i: (i, 0)),
      out_shape=jax.ShapeDtypeStruct((NP, D_H), F32),
  )(hist, y1)

  mp_kernel = _make_scatter_kernel(D_H, NP, C, B, 40)
  p1 = mp_kernel(z1, src3, dst3)

  z2 = pl.pallas_call(
      _make_layer_mid_body(N, RB),
      grid=(n_blocks,),
      in_specs=[part_spec, hist_spec,
                pl.BlockSpec((1, D_H), lambda i: (0, 0)),
                pl.BlockSpec((D_H, D_OUT), lambda i: (0, 0))],
      out_specs=pl.BlockSpec((RB, D_OUT), lambda i: (i, 0)),
      out_shape=jax.ShapeDtypeStruct((NP, D_OUT), F32),
  )(p1, hist, b1.reshape(1, D_H), W2)

  p2 = mp_kernel(z2, src3, dst3)

  out = pl.pallas_call(
      _layer_out_body,
      grid=(n_blocks,),
      in_specs=[pl.BlockSpec((NC, RB, D_OUT), lambda i: (0, i, 0)), hist_spec,
                pl.BlockSpec((1, D_OUT), lambda i: (0, 0))],
      out_specs=pl.BlockSpec((RB, D_OUT), lambda i: (i, 0)),
      out_shape=jax.ShapeDtypeStruct((NP, D_OUT), F32),
  )(p2, hist, b2.reshape(1, D_OUT))

  return out[:N]
